# baseline (speedup 1.0000x reference)
import jax
import jax.numpy as jnp
from jax.experimental import pallas as pl
from jax.experimental.pallas import tpu as pltpu

N = 10000
NUM_SYMPS = 360
RB = 384
NB = 27
NUM_HERBS = 753
NHID = 64
DIM = 64


def _dot(a, b, dn=None):
    if dn is None:
        dn = (((1,), (0,)), ((), ()))
    return jax.lax.dot_general(a, b, dimension_numbers=dn,
                               precision=jax.lax.Precision.DEFAULT,
                               preferred_element_type=jnp.float32)


_DN_T = (((1,), (1,)), ((), ()))


def _mega_kernel(x_ref, w1_ref, b1_ref, w2_ref, b2_ref, wsh_ref, bsh_ref,
                 whc_ref, bhc_ref, adj_ref, sh_ref, hct_ref,
                 s1_ref, s2_ref, ring_ref):
    i = pl.program_id(0)

    @pl.when(i == 0)
    def _():
        s1_ref[...] = _dot(x_ref[...], w1_ref[...])

    @pl.when(i < NB)
    def _():
        h = jnp.maximum(_dot(adj_ref[...], s1_ref[...]) + b1_ref[...], 0.0)
        s2_ref[pl.ds(i * RB, RB), :] = _dot(h, w2_ref[...])

    @pl.when(i >= NB)
    def _():
        k = i - NB
        h2 = _dot(adj_ref[...], s2_ref[:N, :]) + b2_ref[...]

        @pl.when(k == 0)
        def _():
            logits_s = (_dot(h2[:NUM_SYMPS], wsh_ref[...], _DN_T)
                        + bsh_ref[...])
            sh_ref[...] = jax.nn.sigmoid(logits_s)

        logits_t = _dot(whc_ref[...], h2, _DN_T) + bhc_ref[...]
        cur = jax.nn.sigmoid(logits_t)
        w = RB - NUM_SYMPS

        @pl.when(k > 0)
        def _():
            prev = ring_ref[(k - 1) % 2]
            hct_ref[:, :w] = prev[:, NUM_SYMPS:]
            hct_ref[:, w:] = cur[:, :NUM_SYMPS]

        ring_ref[k % 2] = cur


@jax.jit
def kernel(x, adj, W1, b1, W2, b2, Wsh, bsh, Whc, bhc):
    full = lambda shape: pl.BlockSpec(shape, lambda i: (0, 0))

    sh, hct = pl.pallas_call(
        _mega_kernel,
        grid=(2 * NB,),
        in_specs=[
            full((N, x.shape[1])),
            full((x.shape[1], NHID)),
            full((1, NHID)),
            full((NHID, DIM)),
            full((1, DIM)),
            full((NUM_HERBS, DIM)),
            full((1, NUM_HERBS)),
            full((NUM_HERBS, DIM)),
            full((NUM_HERBS, 1)),
            pl.BlockSpec((RB, N),
                         lambda i: (jnp.where(i < NB, i, i - NB), 0)),
        ],
        out_specs=[
            pl.BlockSpec((NUM_SYMPS, NUM_HERBS), lambda i: (0, 0)),
            pl.BlockSpec(
                (NUM_HERBS, RB),
                lambda i: (0, jnp.clip(i - NB - 1, 0,
                                       (N - NUM_SYMPS - 1) // RB))),
        ],
        out_shape=[
            jax.ShapeDtypeStruct((NUM_SYMPS, NUM_HERBS), jnp.float32),
            jax.ShapeDtypeStruct((NUM_HERBS, N - NUM_SYMPS), jnp.float32),
        ],
        scratch_shapes=[
            pltpu.VMEM((N, NHID), jnp.float32),
            pltpu.VMEM((NB * RB, DIM), jnp.float32),
            pltpu.VMEM((2, NUM_HERBS, RB), jnp.float32),
        ],
        compiler_params=pltpu.CompilerParams(
            vmem_limit_bytes=62914560),
    )(x, W1, b1.reshape(1, NHID), W2, b2.reshape(1, DIM),
      Wsh, bsh.reshape(1, NUM_HERBS), Whc, bhc.reshape(NUM_HERBS, 1), adj)

    return (sh, hct.T)
